# Initial kernel scaffold; baseline (speedup 1.0000x reference)
import functools

import jax
import jax.numpy as jnp
from jax import lax
from jax.experimental import pallas as pl
from jax.experimental.pallas import tpu as pltpu
from jax.experimental.pallas import tpu_sc as plsc

N = 10000
E = 320000
D = 128
NC = 2
NS = 16
NW = NC * NS
L = 16

C = 80
EPW_A = E // NW
NCH_A = EPW_A // C
EPW_B = E // NS
NCH_B = EPW_B // C
RPT = N // NS

_MESH = functools.partial(
    plsc.VectorSubcoreMesh, core_axis_name="c", subcore_axis_name="s")


def _hist_body(e0_hbm, e1_hbm, zn_hbm, out_hbm, e0_v, e1_v, h0_v, h1_v):
    c = lax.axis_index("c")
    s = lax.axis_index("s")
    wid = c * NS + s
    base = wid * EPW_A
    pltpu.sync_copy(e0_hbm.at[pl.ds(base, EPW_A)], e0_v)
    pltpu.sync_copy(e1_hbm.at[pl.ds(base, EPW_A)], e1_v)
    pltpu.sync_copy(zn_hbm, h0_v)
    pltpu.sync_copy(zn_hbm, h1_v)
    ones16 = jnp.ones((L,), jnp.float32)

    def hloop(i, carry):
        i0 = e0_v[pl.ds(i * L, L)]
        i1 = e1_v[pl.ds(i * L, L)]
        plsc.addupdate_scatter(h0_v, [i0], ones16)
        plsc.addupdate_scatter(h1_v, [i1], ones16)
        return carry

    lax.fori_loop(0, EPW_A // L, hloop, 0)
    pltpu.sync_copy(h0_v, out_hbm.at[wid, 0])
    pltpu.sync_copy(h1_v, out_hbm.at[wid, 1])


def _hist_call(e0, e1, zn):
    f = pl.kernel(
        _hist_body,
        out_type=jax.ShapeDtypeStruct((NW, 2, N), jnp.float32),
        mesh=_MESH(),
        scratch_types=[
            pltpu.VMEM((EPW_A,), jnp.int32),
            pltpu.VMEM((EPW_A,), jnp.int32),
            pltpu.VMEM((N,), jnp.float32),
            pltpu.VMEM((N,), jnp.float32),
        ],
    )
    return f(e0, e1, zn)


def _make_edge_acc(nch):
    def body(table_hbm, ein_hbm, eout_hbm, znd_hbm, out_hbm,
             iin_v, iout_v, rows_v, acc_sh, sem):
        c = lax.axis_index("c")
        s = lax.axis_index("s")
        pltpu.sync_copy(ein_hbm.at[c, s], iin_v)
        pltpu.sync_copy(eout_hbm.at[c, s], iout_v)
        pltpu.sync_copy(znd_hbm.at[pl.ds(s * RPT, RPT)],
                        acc_sh.at[pl.ds(s * RPT, RPT)])
        plsc.subcore_barrier()

        def aloop(j, carry):
            pltpu.async_copy(table_hbm.at[iin_v.at[j]], rows_v, sem).wait()
            pltpu.sync_copy(rows_v, acc_sh.at[iout_v.at[j]], add=True)
            return carry

        lax.fori_loop(0, nch, aloop, 0)
        plsc.subcore_barrier()
        pltpu.sync_copy(acc_sh.at[pl.ds(s * RPT, RPT)],
                        out_hbm.at[c, pl.ds(s * RPT, RPT)])

    def call(table, ein, eout, znd):
        f = pl.kernel(
            body,
            out_type=jax.ShapeDtypeStruct((NC, N, D), jnp.float32),
            mesh=_MESH(),
            scratch_types=[
                pltpu.VMEM((nch, C), jnp.int32),
                pltpu.VMEM((nch, C), jnp.int32),
                pltpu.VMEM((C, D), jnp.float32),
                pltpu.VMEM_SHARED((N, D), jnp.float32),
                pltpu.SemaphoreType.DMA,
            ],
        )
        return f(table, ein, eout, znd)

    return call


def _prep_tc(hist_t_ref, x_ref, xd_ref, dinv_ref, cnt_ref):
    h = hist_t_ref[...]
    cnt = jnp.sum(h[:, :NW], axis=1, keepdims=True)
    deg = 1.0 + jnp.sum(h[:, NW:], axis=1, keepdims=True)
    dinv = lax.rsqrt(deg)
    cnt_ref[...] = cnt
    dinv_ref[...] = dinv
    xd_ref[...] = x_ref[...] * dinv


def _prep_call(hist_t, x):
    return pl.pallas_call(
        _prep_tc,
        out_shape=[
            jax.ShapeDtypeStruct((N, D), jnp.float32),
            jax.ShapeDtypeStruct((N, 1), jnp.float32),
            jax.ShapeDtypeStruct((N, 1), jnp.float32),
        ],
    )(hist_t, x)


def _mid_tc(acc_ref, x_ref, dinv_ref, w_ref, b_ref, h_ref, t2_ref):
    dinv = dinv_ref[...]
    m = (acc_ref[0] + acc_ref[1]) * dinv + x_ref[...] * (dinv * dinv)
    h = jnp.dot(m, w_ref[...], preferred_element_type=jnp.float32)
    h = jnp.maximum(h + b_ref[...], 0.0)
    h_ref[...] = h
    t2_ref[0] = h
    t2_ref[1] = h * h


def _mid_call(acc, x, dinv, w, b2):
    return pl.pallas_call(
        _mid_tc,
        out_shape=[
            jax.ShapeDtypeStruct((N, D), jnp.float32),
            jax.ShapeDtypeStruct((2, N, D), jnp.float32),
        ],
    )(acc, x, dinv, w, b2)


def _final_tc(sb_ref, h_ref, cnt_ref, gg_ref):
    h = h_ref[...]
    cnt = cnt_ref[...]
    ssum = cnt * h * h - 2.0 * h * sb_ref[0] + sb_ref[1]
    ssum = jnp.maximum(ssum, 0.0)
    gg_ref[...] = jnp.tanh(ssum / jnp.maximum(cnt, 1.0))


def _final_call(sb, h, cnt):
    return pl.pallas_call(
        _final_tc,
        out_shape=jax.ShapeDtypeStruct((N, D), jnp.float32),
    )(sb, h, cnt)


_edge_acc_a = _make_edge_acc(NCH_A)
_edge_acc_b = _make_edge_acc(NCH_B)


def kernel(X, edge_index, W, b):
    e0 = edge_index[0]
    e1 = edge_index[1]
    zn = jnp.zeros((N,), jnp.float32)
    znd = jnp.zeros((N, D), jnp.float32)

    hist = _hist_call(e0, e1, zn)
    hist_t = jnp.transpose(hist, (2, 1, 0)).reshape(N, 2 * NW)
    xd, dinv, cnt = _prep_call(hist_t, X)

    ein_a = e0.reshape(NC, NS, NCH_A, C)
    eout_a = e1.reshape(NC, NS, NCH_A, C)
    acc = _edge_acc_a(xd, ein_a, eout_a, znd)

    h, t2 = _mid_call(acc, X, dinv, W, b.reshape(1, D))

    e1r = e1.reshape(NS, NCH_B, C)
    e0r = e0.reshape(NS, NCH_B, C)
    ein_b = jnp.stack([e1r, e1r + N])
    eout_b = jnp.stack([e0r, e0r])
    sb = _edge_acc_b(t2.reshape(2 * N, D), ein_b, eout_b, znd)

    return _final_call(sb, h, cnt)


# speedup vs baseline: 6.7330x; 6.7330x over previous
import functools

import jax
import jax.numpy as jnp
from jax import lax
from jax.experimental import pallas as pl
from jax.experimental.pallas import tpu as pltpu
from jax.experimental.pallas import tpu_sc as plsc

N = 10000
E = 320000
D = 128
NC = 2
NS = 16
NW = NC * NS
L = 16

C = 80
KB = 8
EP = 327680
EPW_A = EP // NW
NB_A = EPW_A // (KB * C)
EPW_B = EP // NS
NB_B = EPW_B // (KB * C)
EPW_H = E // NW
NP = 10240
RPT = NP // NS

_MESH = functools.partial(
    plsc.VectorSubcoreMesh, core_axis_name="c", subcore_axis_name="s")


def _hist_body(e0_hbm, e1_hbm, zn_hbm, out_hbm, e0_v, e1_v, h0_v, h1_v):
    c = lax.axis_index("c")
    s = lax.axis_index("s")
    wid = c * NS + s
    base = wid * EPW_H
    pltpu.sync_copy(e0_hbm.at[pl.ds(base, EPW_H)], e0_v)
    pltpu.sync_copy(e1_hbm.at[pl.ds(base, EPW_H)], e1_v)
    pltpu.sync_copy(zn_hbm, h0_v)
    pltpu.sync_copy(zn_hbm, h1_v)
    ones16 = jnp.ones((L,), jnp.float32)

    def hloop(i, carry):
        i0 = e0_v[pl.ds(i * L, L)]
        i1 = e1_v[pl.ds(i * L, L)]
        plsc.addupdate_scatter(h0_v, [i0], ones16)
        plsc.addupdate_scatter(h1_v, [i1], ones16)
        return carry

    lax.fori_loop(0, EPW_H // L, hloop, 0)
    pltpu.sync_copy(h0_v, out_hbm.at[wid, 0])
    pltpu.sync_copy(h1_v, out_hbm.at[wid, 1])


def _hist_call(e0, e1, zn):
    f = pl.kernel(
        _hist_body,
        out_type=jax.ShapeDtypeStruct((NW, 2, N), jnp.float32),
        mesh=_MESH(),
        compiler_params=pltpu.CompilerParams(needs_layout_passes=False),
        scratch_types=[
            pltpu.VMEM((EPW_H,), jnp.int32),
            pltpu.VMEM((EPW_H,), jnp.int32),
            pltpu.VMEM((N,), jnp.float32),
            pltpu.VMEM((N,), jnp.float32),
        ],
    )
    return f(e0, e1, zn)


def _make_edge_acc(nb):
    def body(table_hbm, ein_hbm, eout_hbm, znd_hbm, out_hbm,
             iin_v, iout_v, rows_v, acc_sh, sem):
        c = lax.axis_index("c")
        s = lax.axis_index("s")
        pltpu.sync_copy(znd_hbm.at[pl.ds(s * RPT, RPT)],
                        acc_sh.at[pl.ds(s * RPT, RPT)])
        plsc.subcore_barrier()

        def bloop(bi, carry):
            pltpu.sync_copy(ein_hbm.at[c, s, pl.ds(bi * KB, KB)], iin_v)
            pltpu.sync_copy(eout_hbm.at[c, s, pl.ds(bi * KB, KB)], iout_v)

            def aloop(j, carry2):
                pltpu.async_copy(
                    table_hbm.at[iin_v.at[j]], rows_v, sem).wait()
                pltpu.sync_copy(rows_v, acc_sh.at[iout_v.at[j]], add=True)
                return carry2

            lax.fori_loop(0, KB, aloop, 0)
            return carry

        lax.fori_loop(0, nb, bloop, 0)
        plsc.subcore_barrier()
        pltpu.sync_copy(acc_sh.at[pl.ds(s * RPT, RPT)],
                        out_hbm.at[c, pl.ds(s * RPT, RPT)])

    def call(table, ein, eout, znd):
        f = pl.kernel(
            body,
            out_type=jax.ShapeDtypeStruct((NC, NP, D), jnp.float32),
            mesh=_MESH(),
            scratch_types=[
                pltpu.VMEM((KB, C), jnp.int32),
                pltpu.VMEM((KB, C), jnp.int32),
                pltpu.VMEM((C, D), jnp.float32),
                pltpu.VMEM_SHARED((NP, D), jnp.float32),
                pltpu.SemaphoreType.DMA,
            ],
        )
        return f(table, ein, eout, znd)

    return call


def _prep_tc(hist_t_ref, x_ref, xd_ref, dinv_ref, cnt_ref):
    h = hist_t_ref[...]
    cnt = jnp.sum(h[:, :NW], axis=1, keepdims=True)
    deg = 1.0 + jnp.sum(h[:, NW:], axis=1, keepdims=True)
    dinv = lax.rsqrt(deg)
    cnt_ref[...] = cnt
    dinv_ref[...] = dinv
    xd_ref[...] = x_ref[...] * dinv


def _prep_call(hist_t, x):
    return pl.pallas_call(
        _prep_tc,
        out_shape=[
            jax.ShapeDtypeStruct((N, D), jnp.float32),
            jax.ShapeDtypeStruct((N, 1), jnp.float32),
            jax.ShapeDtypeStruct((N, 1), jnp.float32),
        ],
    )(hist_t, x)


def _mid_tc(acc_ref, x_ref, dinv_ref, w_ref, b_ref, h_ref, t2_ref):
    dinv = dinv_ref[...]
    m = (acc_ref[0] + acc_ref[1]) * dinv + x_ref[...] * (dinv * dinv)
    h = jnp.dot(m, w_ref[...], preferred_element_type=jnp.float32)
    h = jnp.maximum(h + b_ref[...], 0.0)
    h_ref[...] = h
    t2_ref[0] = h
    t2_ref[1] = h * h


def _mid_call(acc, x, dinv, w, b2):
    return pl.pallas_call(
        _mid_tc,
        out_shape=[
            jax.ShapeDtypeStruct((N, D), jnp.float32),
            jax.ShapeDtypeStruct((2, N, D), jnp.float32),
        ],
    )(acc, x, dinv, w, b2)


def _final_tc(sb_ref, h_ref, cnt_ref, gg_ref):
    h = h_ref[...]
    cnt = cnt_ref[...]
    ssum = cnt * h * h - 2.0 * h * sb_ref[0] + sb_ref[1]
    ssum = jnp.maximum(ssum, 0.0)
    gg_ref[...] = jnp.tanh(ssum / jnp.maximum(cnt, 1.0))


def _final_call(sb, h, cnt):
    return pl.pallas_call(
        _final_tc,
        out_shape=jax.ShapeDtypeStruct((N, D), jnp.float32),
    )(sb, h, cnt)


_edge_acc_a = _make_edge_acc(NB_A)
_edge_acc_b = _make_edge_acc(NB_B)


def kernel(X, edge_index, W, b):
    e0 = edge_index[0]
    e1 = edge_index[1]
    zn = jnp.zeros((N,), jnp.float32)
    znd = jnp.zeros((NP, D), jnp.float32)

    hist = _hist_call(e0, e1, zn)
    hist_t = jnp.transpose(hist, (2, 1, 0)).reshape(N, 2 * NW)
    xd, dinv, cnt = _prep_call(hist_t, X)

    pad_in = jnp.zeros((EP - E,), jnp.int32)
    pad_out = jnp.full((EP - E,), N, jnp.int32)
    e0_in = jnp.concatenate([e0, pad_in])
    e1_in = jnp.concatenate([e1, pad_in])
    e0_out = jnp.concatenate([e0, pad_out])
    e1_out = jnp.concatenate([e1, pad_out])

    ein_a = e0_in.reshape(NC, NS, NB_A * KB, C)
    eout_a = e1_out.reshape(NC, NS, NB_A * KB, C)
    acc = _edge_acc_a(xd, ein_a, eout_a, znd)[:, :N]

    h, t2 = _mid_call(acc, X, dinv, W, b.reshape(1, D))

    e1r = e1_in.reshape(NS, NB_B * KB, C)
    e0r = e0_out.reshape(NS, NB_B * KB, C)
    ein_b = jnp.stack([e1r, e1r + N])
    eout_b = jnp.stack([e0r, e0r])
    sb = _edge_acc_b(t2.reshape(2 * N, D), ein_b, eout_b, znd)[:, :N]

    return _final_call(sb, h, cnt)


# speedup vs baseline: 8.1388x; 1.2088x over previous
import functools

import jax
import jax.numpy as jnp
from jax import lax
from jax.experimental import pallas as pl
from jax.experimental.pallas import tpu as pltpu
from jax.experimental.pallas import tpu_sc as plsc

N = 10000
E = 320000
D = 128
NC = 2
NS = 16
NW = NC * NS
L = 16

C = 80
KB = 8
EP = 327680
EPW_A = EP // NW
NB_A = EPW_A // (KB * C)
EPW_B = EP // NS
NB_B = EPW_B // (KB * C)
EPW_H = E // NW
NP = 10240
RPT = NP // NS

_MESH = functools.partial(
    plsc.VectorSubcoreMesh, core_axis_name="c", subcore_axis_name="s")


def _hist_body(e0_hbm, e1_hbm, zn_hbm, out_hbm, e0_v, e1_v, h0_v, h1_v):
    c = lax.axis_index("c")
    s = lax.axis_index("s")
    wid = c * NS + s
    base = wid * EPW_H
    pltpu.sync_copy(e0_hbm.at[pl.ds(base, EPW_H)], e0_v)
    pltpu.sync_copy(e1_hbm.at[pl.ds(base, EPW_H)], e1_v)
    pltpu.sync_copy(zn_hbm, h0_v)
    pltpu.sync_copy(zn_hbm, h1_v)
    ones16 = jnp.ones((L,), jnp.float32)

    def hloop(i, carry):
        i0 = e0_v[pl.ds(i * L, L)]
        i1 = e1_v[pl.ds(i * L, L)]
        plsc.addupdate_scatter(h0_v, [i0], ones16)
        plsc.addupdate_scatter(h1_v, [i1], ones16)
        return carry

    lax.fori_loop(0, EPW_H // L, hloop, 0)
    pltpu.sync_copy(h0_v, out_hbm.at[wid, 0])
    pltpu.sync_copy(h1_v, out_hbm.at[wid, 1])


def _hist_call(e0, e1, zn):
    f = pl.kernel(
        _hist_body,
        out_type=jax.ShapeDtypeStruct((NW, 2, N), jnp.float32),
        mesh=_MESH(),
        compiler_params=pltpu.CompilerParams(needs_layout_passes=False),
        scratch_types=[
            pltpu.VMEM((EPW_H,), jnp.int32),
            pltpu.VMEM((EPW_H,), jnp.int32),
            pltpu.VMEM((N,), jnp.float32),
            pltpu.VMEM((N,), jnp.float32),
        ],
    )
    return f(e0, e1, zn)


def _make_edge_acc(nb):
    def body(table_hbm, ein_hbm, eout_hbm, znd_hbm, out_hbm,
             iin_v, iout_v, rows_v, acc_sh, gsem0, gsem1, isem):
        c = lax.axis_index("c")
        s = lax.axis_index("s")
        pltpu.sync_copy(znd_hbm.at[pl.ds(s * RPT, RPT)],
                        acc_sh.at[pl.ds(s * RPT, RPT)])
        plsc.subcore_barrier()

        pltpu.sync_copy(ein_hbm.at[c, s, pl.ds(0, KB)], iin_v.at[0])
        pltpu.sync_copy(eout_hbm.at[c, s, pl.ds(0, KB)], iout_v.at[0])

        def bloop(bi, carry):
            pb = lax.rem(bi, 2)

            @pl.when(bi + 1 < nb)
            def _prefetch_idx():
                nxt = (bi + 1) * KB
                pltpu.async_copy(ein_hbm.at[c, s, pl.ds(nxt, KB)],
                                 iin_v.at[1 - pb], isem)
                pltpu.async_copy(eout_hbm.at[c, s, pl.ds(nxt, KB)],
                                 iout_v.at[1 - pb], isem)

            gsems = (gsem0, gsem1)
            d = pltpu.async_copy(
                table_hbm.at[iin_v.at[pb, 0]], rows_v.at[0], gsems[0])
            for j in range(KB):
                p = j % 2
                if j + 1 < KB:
                    dn = pltpu.async_copy(
                        table_hbm.at[iin_v.at[pb, j + 1]],
                        rows_v.at[1 - p], gsems[(j + 1) % 2])
                d.wait()
                pltpu.sync_copy(rows_v.at[p],
                                acc_sh.at[iout_v.at[pb, j]], add=True)
                if j + 1 < KB:
                    d = dn

            @pl.when(bi + 1 < nb)
            def _wait_idx():
                nxt = (bi + 1) * KB
                pltpu.make_async_copy(ein_hbm.at[c, s, pl.ds(nxt, KB)],
                                      iin_v.at[1 - pb], isem).wait()
                pltpu.make_async_copy(eout_hbm.at[c, s, pl.ds(nxt, KB)],
                                      iout_v.at[1 - pb], isem).wait()

            return carry

        lax.fori_loop(0, nb, bloop, 0)
        plsc.subcore_barrier()
        pltpu.sync_copy(acc_sh.at[pl.ds(s * RPT, RPT)],
                        out_hbm.at[c, pl.ds(s * RPT, RPT)])

    def call(table, ein, eout, znd):
        f = pl.kernel(
            body,
            out_type=jax.ShapeDtypeStruct((NC, NP, D), jnp.float32),
            mesh=_MESH(),
            scratch_types=[
                pltpu.VMEM((2, KB, C), jnp.int32),
                pltpu.VMEM((2, KB, C), jnp.int32),
                pltpu.VMEM((2, C, D), jnp.float32),
                pltpu.VMEM_SHARED((NP, D), jnp.float32),
                pltpu.SemaphoreType.DMA,
                pltpu.SemaphoreType.DMA,
                pltpu.SemaphoreType.DMA,
            ],
        )
        return f(table, ein, eout, znd)

    return call


def _prep_tc(hist_t_ref, x_ref, xd_ref, dinv_ref, cnt_ref):
    h = hist_t_ref[...]
    cnt = jnp.sum(h[:, :NW], axis=1, keepdims=True)
    deg = 1.0 + jnp.sum(h[:, NW:], axis=1, keepdims=True)
    dinv = lax.rsqrt(deg)
    cnt_ref[...] = cnt
    dinv_ref[...] = dinv
    xd_ref[...] = x_ref[...] * dinv


def _prep_call(hist_t, x):
    return pl.pallas_call(
        _prep_tc,
        out_shape=[
            jax.ShapeDtypeStruct((N, D), jnp.float32),
            jax.ShapeDtypeStruct((N, 1), jnp.float32),
            jax.ShapeDtypeStruct((N, 1), jnp.float32),
        ],
    )(hist_t, x)


def _mid_tc(acc_ref, x_ref, dinv_ref, w_ref, b_ref, h_ref, t2_ref):
    dinv = dinv_ref[...]
    m = (acc_ref[0] + acc_ref[1]) * dinv + x_ref[...] * (dinv * dinv)
    h = jnp.dot(m, w_ref[...], preferred_element_type=jnp.float32)
    h = jnp.maximum(h + b_ref[...], 0.0)
    h_ref[...] = h
    t2_ref[0] = h
    t2_ref[1] = h * h


def _mid_call(acc, x, dinv, w, b2):
    return pl.pallas_call(
        _mid_tc,
        out_shape=[
            jax.ShapeDtypeStruct((N, D), jnp.float32),
            jax.ShapeDtypeStruct((2, N, D), jnp.float32),
        ],
    )(acc, x, dinv, w, b2)


def _final_tc(sb_ref, h_ref, cnt_ref, gg_ref):
    h = h_ref[...]
    cnt = cnt_ref[...]
    ssum = cnt * h * h - 2.0 * h * sb_ref[0] + sb_ref[1]
    ssum = jnp.maximum(ssum, 0.0)
    gg_ref[...] = jnp.tanh(ssum / jnp.maximum(cnt, 1.0))


def _final_call(sb, h, cnt):
    return pl.pallas_call(
        _final_tc,
        out_shape=jax.ShapeDtypeStruct((N, D), jnp.float32),
    )(sb, h, cnt)


_edge_acc_a = _make_edge_acc(NB_A)
_edge_acc_b = _make_edge_acc(NB_B)


def kernel(X, edge_index, W, b):
    e0 = edge_index[0]
    e1 = edge_index[1]
    zn = jnp.zeros((N,), jnp.float32)
    znd = jnp.zeros((NP, D), jnp.float32)

    hist = _hist_call(e0, e1, zn)
    hist_t = jnp.transpose(hist, (2, 1, 0)).reshape(N, 2 * NW)
    xd, dinv, cnt = _prep_call(hist_t, X)

    pad_in = jnp.zeros((EP - E,), jnp.int32)
    pad_out = jnp.full((EP - E,), N, jnp.int32)
    e0_in = jnp.concatenate([e0, pad_in])
    e1_in = jnp.concatenate([e1, pad_in])
    e0_out = jnp.concatenate([e0, pad_out])
    e1_out = jnp.concatenate([e1, pad_out])

    ein_a = e0_in.reshape(NC, NS, NB_A * KB, C)
    eout_a = e1_out.reshape(NC, NS, NB_A * KB, C)
    acc = _edge_acc_a(xd, ein_a, eout_a, znd)[:, :N]

    h, t2 = _mid_call(acc, X, dinv, W, b.reshape(1, D))

    e1r = e1_in.reshape(NS, NB_B * KB, C)
    e0r = e0_out.reshape(NS, NB_B * KB, C)
    ein_b = jnp.stack([e1r, e1r + N])
    eout_b = jnp.stack([e0r, e0r])
    sb = _edge_acc_b(t2.reshape(2 * N, D), ein_b, eout_b, znd)[:, :N]

    return _final_call(sb, h, cnt)


# speedup vs baseline: 20.0945x; 2.4690x over previous
import functools

import jax
import jax.numpy as jnp
from jax import lax
from jax.experimental import pallas as pl
from jax.experimental.pallas import tpu as pltpu
from jax.experimental.pallas import tpu_sc as plsc

N = 10000
E = 320000
D = 128
NC = 2
NS = 16
NW = NC * NS
L = 16

C = 80
KB = 8
EP = 327680
EPW_A = EP // NW
NB_A = EPW_A // (KB * C)
EPW_B = EP // NS
NB_B = EPW_B // (KB * C)
EPW_H = E // NW
NP = 10240
RPT = NP // NS

_MESH = functools.partial(
    plsc.VectorSubcoreMesh, core_axis_name="c", subcore_axis_name="s")


def _hist_body(e0_hbm, e1_hbm, zn_hbm, out_hbm, e0_v, e1_v, h0_v, h1_v):
    c = lax.axis_index("c")
    s = lax.axis_index("s")
    wid = c * NS + s
    base = wid * EPW_H
    pltpu.sync_copy(e0_hbm.at[pl.ds(base, EPW_H)], e0_v)
    pltpu.sync_copy(e1_hbm.at[pl.ds(base, EPW_H)], e1_v)
    pltpu.sync_copy(zn_hbm, h0_v)
    pltpu.sync_copy(zn_hbm, h1_v)
    ones16 = jnp.ones((L,), jnp.float32)

    def hloop(i, carry):
        i0 = e0_v[pl.ds(i * L, L)]
        i1 = e1_v[pl.ds(i * L, L)]
        plsc.addupdate_scatter(h0_v, [i0], ones16)
        plsc.addupdate_scatter(h1_v, [i1], ones16)
        return carry

    lax.fori_loop(0, EPW_H // L, hloop, 0)
    pltpu.sync_copy(h0_v, out_hbm.at[wid, 0])
    pltpu.sync_copy(h1_v, out_hbm.at[wid, 1])


def _hist_call(e0, e1, zn):
    f = pl.kernel(
        _hist_body,
        out_type=jax.ShapeDtypeStruct((NW, 2, N), jnp.float32),
        mesh=_MESH(),
        compiler_params=pltpu.CompilerParams(needs_layout_passes=False),
        scratch_types=[
            pltpu.VMEM((EPW_H,), jnp.int32),
            pltpu.VMEM((EPW_H,), jnp.int32),
            pltpu.VMEM((N,), jnp.float32),
            pltpu.VMEM((N,), jnp.float32),
        ],
    )
    return f(e0, e1, zn)


def _make_edge_acc(nb):
    def body(table_hbm, ein_hbm, eout_hbm, znd_hbm, out_hbm,
             iin_v, iout_v, rows_v, acc_sh, gsem0, gsem1, isem):
        c = lax.axis_index("c")
        s = lax.axis_index("s")
        pltpu.sync_copy(znd_hbm.at[pl.ds(s * RPT, RPT)],
                        acc_sh.at[pl.ds(s * RPT, RPT)])
        plsc.subcore_barrier()

        pltpu.sync_copy(ein_hbm.at[c, s, pl.ds(0, KB)], iin_v.at[0])
        pltpu.sync_copy(eout_hbm.at[c, s, pl.ds(0, KB)], iout_v.at[0])

        def bloop(bi, carry):
            pb = lax.rem(bi, 2)

            @pl.when(bi + 1 < nb)
            def _prefetch_idx():
                nxt = (bi + 1) * KB
                pltpu.async_copy(ein_hbm.at[c, s, pl.ds(nxt, KB)],
                                 iin_v.at[1 - pb], isem)
                pltpu.async_copy(eout_hbm.at[c, s, pl.ds(nxt, KB)],
                                 iout_v.at[1 - pb], isem)

            gsems = (gsem0, gsem1)
            d = pltpu.async_copy(
                table_hbm.at[iin_v.at[pb, 0]], rows_v.at[0], gsems[0])
            for j in range(KB):
                p = j % 2
                if j + 1 < KB:
                    dn = pltpu.async_copy(
                        table_hbm.at[iin_v.at[pb, j + 1]],
                        rows_v.at[1 - p], gsems[(j + 1) % 2])
                d.wait()
                pltpu.sync_copy(rows_v.at[p],
                                acc_sh.at[iout_v.at[pb, j]], add=True)
                if j + 1 < KB:
                    d = dn

            @pl.when(bi + 1 < nb)
            def _wait_idx():
                nxt = (bi + 1) * KB
                pltpu.make_async_copy(ein_hbm.at[c, s, pl.ds(nxt, KB)],
                                      iin_v.at[1 - pb], isem).wait()
                pltpu.make_async_copy(eout_hbm.at[c, s, pl.ds(nxt, KB)],
                                      iout_v.at[1 - pb], isem).wait()

            return carry

        lax.fori_loop(0, nb, bloop, 0)
        plsc.subcore_barrier()
        pltpu.sync_copy(acc_sh.at[pl.ds(s * RPT, RPT)],
                        out_hbm.at[c, pl.ds(s * RPT, RPT)])

    def call(table, ein, eout, znd):
        f = pl.kernel(
            body,
            out_type=jax.ShapeDtypeStruct((NC, NP, D), jnp.float32),
            mesh=_MESH(),
            scratch_types=[
                pltpu.VMEM((2, KB, C), jnp.int32),
                pltpu.VMEM((2, KB, C), jnp.int32),
                pltpu.VMEM((2, C, D), jnp.float32),
                pltpu.VMEM_SHARED((NP, D), jnp.float32),
                pltpu.SemaphoreType.DMA,
                pltpu.SemaphoreType.DMA,
                pltpu.SemaphoreType.DMA,
            ],
        )
        return f(table, ein, eout, znd)

    return call


def _prep_tc(hist_t_ref, x_ref, xd_ref, dinv_ref, cnt_ref):
    h = hist_t_ref[...]
    cnt = jnp.sum(h[:, :NW], axis=1, keepdims=True)
    deg = 1.0 + jnp.sum(h[:, NW:], axis=1, keepdims=True)
    dinv = lax.rsqrt(deg)
    cnt_ref[...] = cnt
    dinv_ref[...] = dinv
    xd_ref[...] = x_ref[...] * dinv


def _prep_call(hist_t, x):
    return pl.pallas_call(
        _prep_tc,
        out_shape=[
            jax.ShapeDtypeStruct((N, D), jnp.float32),
            jax.ShapeDtypeStruct((N, 1), jnp.float32),
            jax.ShapeDtypeStruct((N, 1), jnp.float32),
        ],
    )(hist_t, x)


def _mid_tc(acc_ref, x_ref, dinv_ref, w_ref, b_ref, h_ref, t2_ref):
    dinv = dinv_ref[...]
    m = (acc_ref[0] + acc_ref[1]) * dinv + x_ref[...] * (dinv * dinv)
    h = jnp.dot(m, w_ref[...], preferred_element_type=jnp.float32)
    h = jnp.maximum(h + b_ref[...], 0.0)
    h_ref[...] = h
    t2_ref[0] = h
    t2_ref[1] = h * h


def _mid_call(acc, x, dinv, w, b2):
    return pl.pallas_call(
        _mid_tc,
        out_shape=[
            jax.ShapeDtypeStruct((N, D), jnp.float32),
            jax.ShapeDtypeStruct((2, N, D), jnp.float32),
        ],
    )(acc, x, dinv, w, b2)


def _final_tc(sb_ref, h_ref, cnt_ref, gg_ref):
    h = h_ref[...]
    cnt = cnt_ref[...]
    ssum = cnt * h * h - 2.0 * h * sb_ref[0] + sb_ref[1]
    ssum = jnp.maximum(ssum, 0.0)
    gg_ref[...] = jnp.tanh(ssum / jnp.maximum(cnt, 1.0))


def _final_call(sb, h, cnt):
    return pl.pallas_call(
        _final_tc,
        out_shape=jax.ShapeDtypeStruct((N, D), jnp.float32),
    )(sb, h, cnt)


_edge_acc_a = _make_edge_acc(NB_A)
_edge_acc_b = _make_edge_acc(NB_B)


def kernel(X, edge_index, W, b):
    e0 = edge_index[0]
    e1 = edge_index[1]
    zn = jnp.zeros((N,), jnp.float32)
    znd = jnp.zeros((NP, D), jnp.float32)

    hist = _hist_call(e0, e1, zn)
    hist_t = jnp.transpose(hist, (2, 1, 0)).reshape(N, 2 * NW)
    xd, dinv, cnt = _prep_call(hist_t, X)

    pad_idx = jnp.arange(EP - E, dtype=jnp.int32)
    pad_in = pad_idx % N
    pad_out = N + pad_idx % (NP - N)
    e0_in = jnp.concatenate([e0, pad_in])
    e1_in = jnp.concatenate([e1, pad_in])
    e0_out = jnp.concatenate([e0, pad_out])
    e1_out = jnp.concatenate([e1, pad_out])

    ein_a = e0_in.reshape(NC, NS, NB_A * KB, C)
    eout_a = e1_out.reshape(NC, NS, NB_A * KB, C)
    acc = _edge_acc_a(xd, ein_a, eout_a, znd)[:, :N]

    h, t2 = _mid_call(acc, X, dinv, W, b.reshape(1, D))

    e1r = e1_in.reshape(NS, NB_B * KB, C)
    e0r = e0_out.reshape(NS, NB_B * KB, C)
    ein_b = jnp.stack([e1r, e1r + N])
    eout_b = jnp.stack([e0r, e0r])
    sb = _edge_acc_b(t2.reshape(2 * N, D), ein_b, eout_b, znd)[:, :N]

    return _final_call(sb, h, cnt)


# speedup vs baseline: 21.4605x; 1.0680x over previous
import functools

import jax
import jax.numpy as jnp
from jax import lax
from jax.experimental import pallas as pl
from jax.experimental.pallas import tpu as pltpu
from jax.experimental.pallas import tpu_sc as plsc

N = 10000
E = 320000
D = 128
NC = 2
NS = 16
NW = NC * NS
L = 16

C = 80
KB = 8
EP = 327680
NCH = EP // C
EPW_A = EP // NW
NB_A = EPW_A // (KB * C)
EPW_B = EP // NS
NB_B = EPW_B // (KB * C)
EPW_H = E // NW
NP = 10240
RPT = NP // NS

_MESH = functools.partial(
    plsc.VectorSubcoreMesh, core_axis_name="c", subcore_axis_name="s")


def _hist_body(ei_hbm, zn_hbm, out_hbm, e0_v, e1_v, h0_v, h1_v):
    c = lax.axis_index("c")
    s = lax.axis_index("s")
    wid = c * NS + s
    base = wid * EPW_H
    pltpu.sync_copy(ei_hbm.at[pl.ds(base, EPW_H)], e0_v)
    pltpu.sync_copy(ei_hbm.at[pl.ds(E + base, EPW_H)], e1_v)
    pltpu.sync_copy(zn_hbm, h0_v)
    pltpu.sync_copy(zn_hbm, h1_v)
    ones16 = jnp.ones((L,), jnp.float32)

    def hloop(i, carry):
        i0 = e0_v[pl.ds(i * L, L)]
        i1 = e1_v[pl.ds(i * L, L)]
        plsc.addupdate_scatter(h0_v, [i0], ones16)
        plsc.addupdate_scatter(h1_v, [i1], ones16)
        return carry

    lax.fori_loop(0, EPW_H // L, hloop, 0)
    pltpu.sync_copy(h0_v, out_hbm.at[wid, 0])
    pltpu.sync_copy(h1_v, out_hbm.at[wid, 1])


def _hist_call(edge_index, zn):
    f = pl.kernel(
        _hist_body,
        out_type=jax.ShapeDtypeStruct((NW, 2, N), jnp.float32),
        mesh=_MESH(),
        compiler_params=pltpu.CompilerParams(needs_layout_passes=False),
        scratch_types=[
            pltpu.VMEM((EPW_H,), jnp.int32),
            pltpu.VMEM((EPW_H,), jnp.int32),
            pltpu.VMEM((N,), jnp.float32),
            pltpu.VMEM((N,), jnp.float32),
        ],
    )
    return f(edge_index, zn)


def _make_edge_acc(nb, gdim, sdim, split_cores, core_offset):
    def body(table_hbm, eip_hbm, znd_hbm, out_hbm,
             iin_v, iout_v, rows_v, acc_sh, gsem0, gsem1, isem):
        c = lax.axis_index("c")
        s = lax.axis_index("s")
        pltpu.sync_copy(znd_hbm.at[pl.ds(s * RPT, RPT)],
                        acc_sh.at[pl.ds(s * RPT, RPT)])
        plsc.subcore_barrier()

        if split_cores:
            crow = (c * NS + s) * (nb * KB)
        else:
            crow = s * (nb * KB)

        def load_idx(rowbase, slot):
            pltpu.async_copy(eip_hbm.at[gdim, pl.ds(rowbase, KB)],
                             iin_v.at[slot], isem)
            pltpu.async_copy(eip_hbm.at[sdim, pl.ds(rowbase, KB)],
                             iout_v.at[slot], isem)

        def wait_idx(rowbase, slot):
            pltpu.make_async_copy(eip_hbm.at[gdim, pl.ds(rowbase, KB)],
                                  iin_v.at[slot], isem).wait()
            pltpu.make_async_copy(eip_hbm.at[sdim, pl.ds(rowbase, KB)],
                                  iout_v.at[slot], isem).wait()

        def add_core_offset(slot):
            if core_offset:
                off = jnp.broadcast_to((c * core_offset).astype(jnp.int32),
                                       (L,))
                for r in range(KB):
                    for k in range(C // L):
                        iin_v[slot, r, pl.ds(k * L, L)] = (
                            iin_v[slot, r, pl.ds(k * L, L)] + off)

        load_idx(crow, 0)
        wait_idx(crow, 0)
        add_core_offset(0)

        def bloop(bi, carry):
            pb = lax.rem(bi, 2)

            @pl.when(bi + 1 < nb)
            def _prefetch_idx():
                load_idx(crow + (bi + 1) * KB, 1 - pb)

            gsems = (gsem0, gsem1)
            d = pltpu.async_copy(
                table_hbm.at[iin_v.at[pb, 0]], rows_v.at[0], gsems[0])
            for j in range(KB):
                p = j % 2
                if j + 1 < KB:
                    dn = pltpu.async_copy(
                        table_hbm.at[iin_v.at[pb, j + 1]],
                        rows_v.at[1 - p], gsems[(j + 1) % 2])
                d.wait()
                pltpu.sync_copy(rows_v.at[p],
                                acc_sh.at[iout_v.at[pb, j]], add=True)
                if j + 1 < KB:
                    d = dn

            @pl.when(bi + 1 < nb)
            def _wait_idx():
                wait_idx(crow + (bi + 1) * KB, 1 - pb)

            add_core_offset(1 - pb)
            return carry

        lax.fori_loop(0, nb, bloop, 0)
        plsc.subcore_barrier()
        pltpu.sync_copy(acc_sh.at[pl.ds(s * RPT, RPT)],
                        out_hbm.at[c, pl.ds(s * RPT, RPT)])

    def call(table, eip, znd):
        f = pl.kernel(
            body,
            out_type=jax.ShapeDtypeStruct((NC, NP, D), jnp.float32),
            mesh=_MESH(),
            scratch_types=[
                pltpu.VMEM((2, KB, C), jnp.int32),
                pltpu.VMEM((2, KB, C), jnp.int32),
                pltpu.VMEM((2, C, D), jnp.float32),
                pltpu.VMEM_SHARED((NP, D), jnp.float32),
                pltpu.SemaphoreType.DMA,
                pltpu.SemaphoreType.DMA,
                pltpu.SemaphoreType.DMA,
            ],
        )
        return f(table, eip, znd)

    return call


_edge_acc_a = _make_edge_acc(NB_A, 0, 1, True, 0)
_edge_acc_b = _make_edge_acc(NB_B, 1, 0, False, N)


def _dinv_of(h):
    return lax.rsqrt(1.0 + jnp.sum(h[:, NW:], axis=1, keepdims=True))


def _prep_tc(hist_t_ref, x_ref, xd_ref):
    xd_ref[...] = x_ref[...] * _dinv_of(hist_t_ref[...])


def _prep_call(hist_t, x):
    return pl.pallas_call(
        _prep_tc,
        out_shape=jax.ShapeDtypeStruct((N, D), jnp.float32),
    )(hist_t, x)


def _mid_tc(acc_ref, x_ref, hist_t_ref, w_ref, b_ref, h_ref, t2_ref):
    dinv = _dinv_of(hist_t_ref[...])
    m = ((acc_ref[0, :N] + acc_ref[1, :N]) * dinv
         + x_ref[...] * (dinv * dinv))
    h = jnp.dot(m, w_ref[...], preferred_element_type=jnp.float32)
    h = jnp.maximum(h + b_ref[...], 0.0)
    h_ref[...] = h
    t2_ref[0] = h
    t2_ref[1] = h * h


def _mid_call(acc, x, hist_t, w, b2):
    return pl.pallas_call(
        _mid_tc,
        out_shape=[
            jax.ShapeDtypeStruct((N, D), jnp.float32),
            jax.ShapeDtypeStruct((2, N, D), jnp.float32),
        ],
    )(acc, x, hist_t, w, b2)


def _final_tc(sb_ref, h_ref, hist_t_ref, gg_ref):
    h = h_ref[...]
    cnt = jnp.sum(hist_t_ref[...][:, :NW], axis=1, keepdims=True)
    ssum = cnt * h * h - 2.0 * h * sb_ref[0, :N] + sb_ref[1, :N]
    ssum = jnp.maximum(ssum, 0.0)
    gg_ref[...] = jnp.tanh(ssum / jnp.maximum(cnt, 1.0))


def _final_call(sb, h, hist_t):
    return pl.pallas_call(
        _final_tc,
        out_shape=jax.ShapeDtypeStruct((N, D), jnp.float32),
    )(sb, h, hist_t)


def kernel(X, edge_index, W, b):
    zn = jnp.zeros((N,), jnp.float32)
    znd = jnp.zeros((NP, D), jnp.float32)

    hist = _hist_call(edge_index.reshape(2 * E), zn)
    hist_t = jnp.transpose(hist, (2, 1, 0)).reshape(N, 2 * NW)
    xd = _prep_call(hist_t, X)

    pad_idx = jnp.arange(EP - E, dtype=jnp.int32)
    pad_lo = pad_idx % N
    pad_hi = N + pad_idx % (NP - N)
    eip_a = jnp.concatenate(
        [edge_index, jnp.stack([pad_lo, pad_hi])], axis=1).reshape(2, NCH, C)
    eip_b = jnp.concatenate(
        [edge_index, jnp.stack([pad_hi, pad_lo])], axis=1).reshape(2, NCH, C)

    acc = _edge_acc_a(xd, eip_a, znd)
    h, t2 = _mid_call(acc, X, hist_t, W, b.reshape(1, D))
    sb = _edge_acc_b(t2.reshape(2 * N, D), eip_b, znd)
    return _final_call(sb, h, hist_t)
